# baseline (speedup 1.0000x reference)
import functools

import jax
import jax.numpy as jnp
from jax import lax
from jax.experimental import pallas as pl
from jax.experimental.pallas import tpu as pltpu
from jax.experimental.pallas import tpu_sc as plsc

_B = 1
_T = 2048
_BT = _B * _T
_D = 1024
_H = 4096
_E = 8
_K = 2

_TR = 128
_NTMAX = _BT * _K // _TR + _E
_ROWS = _NTMAX * _TR
_HBLK = 1024
_NH = _H // _HBLK
_W = _NH * _NTMAX

_NC = 2
_NS = 16
_NW = _NC * _NS
_A = _BT * _K
_APW = _A // _NW
_RCH = 32
_NCH = _APW // _RCH
_TPW = _BT // _NW
_TCH = 16
_LCH = _D // 16

_sc_mesh = plsc.VectorSubcoreMesh(core_axis_name="c", subcore_axis_name="s")


def _router_body(x_ref, g_ref, idx_ref, w_ref):
    logits = lax.dot_general(x_ref[...], g_ref[...],
                             (((1,), (1,)), ((), ())),
                             preferred_element_type=jnp.float32)
    col = lax.broadcasted_iota(jnp.int32, (_BT, _E), 1)
    m1 = jnp.max(logits, axis=1, keepdims=True)
    i1 = jnp.min(jnp.where(logits == m1, col, _E), axis=1, keepdims=True)
    l2 = jnp.where(col == i1, -jnp.inf, logits)
    m2 = jnp.max(l2, axis=1, keepdims=True)
    i2 = jnp.min(jnp.where(l2 == m2, col, _E), axis=1, keepdims=True)
    w1 = 1.0 / (1.0 + jnp.exp(m2 - m1))
    idx_ref[:, 0:1] = i1
    idx_ref[:, 1:2] = i2
    w_ref[:, 0:1] = w1
    w_ref[:, 1:2] = 1.0 - w1


def _router(x_flat, gate_inp):
    return pl.pallas_call(
        _router_body,
        out_shape=(jax.ShapeDtypeStruct((_BT, _K), jnp.int32),
                   jax.ShapeDtypeStruct((_BT, _K), jnp.float32)),
    )(x_flat, gate_inp)


@functools.partial(
    pl.kernel,
    out_type=jax.ShapeDtypeStruct((_ROWS, _D), jnp.float32),
    mesh=_sc_mesh,
    scratch_types=[
        pltpu.VMEM((_NCH, _RCH), jnp.int32),
        pltpu.VMEM((_NCH, _RCH), jnp.int32),
        pltpu.VMEM((_RCH, _D), jnp.float32),
        pltpu.SemaphoreType.DMA,
        pltpu.SemaphoreType.DMA,
    ],
)
def _dispatch(x_hbm, tok_hbm, pos_hbm, xs_hbm, tok_v, pos_v, rows_v, sem, sem2):
    wid = lax.axis_index("s") * _NC + lax.axis_index("c")
    pltpu.sync_copy(tok_hbm.at[wid], tok_v)
    pltpu.sync_copy(pos_hbm.at[wid], pos_v)
    for c in range(_NCH):
        pltpu.async_copy(x_hbm.at[tok_v.at[c]], rows_v, sem).wait()
        pltpu.async_copy(rows_v, xs_hbm.at[pos_v.at[c]], sem2).wait()


@functools.partial(
    pl.kernel,
    out_type=jax.ShapeDtypeStruct((_BT, _D), jnp.float32),
    mesh=_sc_mesh,
    scratch_types=[
        pltpu.VMEM((_NCH, 2 * _TCH), jnp.int32),
        pltpu.VMEM((_NCH * 2 * _TCH, 16), jnp.float32),
        pltpu.VMEM((2 * _TCH, _D), jnp.float32),
        pltpu.VMEM((_TCH, _D), jnp.float32),
        pltpu.SemaphoreType.DMA,
    ],
)
def _combine(y_hbm, pos_hbm, w_hbm, out_hbm, pos_v, w_v, rows_v, out_v, sem):
    wid = lax.axis_index("s") * _NC + lax.axis_index("c")
    pltpu.sync_copy(pos_hbm.at[wid], pos_v)
    pltpu.sync_copy(w_hbm.at[wid], w_v)
    for c in range(_NCH):
        pltpu.async_copy(y_hbm.at[pos_v.at[c]], rows_v, sem).wait()

        def token_loop(j, carry):
            w0 = w_v[c * 2 * _TCH + 2 * j]
            w1 = w_v[c * 2 * _TCH + 2 * j + 1]

            def lane_loop(l8, carry2):
                for q in range(8):
                    sl = pl.ds((l8 * 8 + q) * 16, 16)
                    a0 = rows_v[2 * j, sl]
                    a1 = rows_v[2 * j + 1, sl]
                    out_v[j, sl] = w0 * a0 + w1 * a1
                return carry2

            return lax.fori_loop(0, _LCH // 8, lane_loop, carry)

        lax.fori_loop(0, _TCH, token_loop, 0)
        pltpu.sync_copy(out_v, out_hbm.at[pl.ds(wid * _TPW + c * _TCH, _TCH)])


def _ffn_body(rt_ref, jw_ref, vl_ref, ew_ref, x_ref, wu_ref, wg_ref,
              wd_ref, out_ref):
    w = pl.program_id(0)
    rt = rt_ref[w]
    jj = jw_ref[w]

    @pl.when(vl_ref[w] == 1)
    def _():
        xt = x_ref[...]
        uvec = lax.dot_general(xt, wu_ref[0], (((1,), (1,)), ((), ())),
                               preferred_element_type=jnp.float32)
        gvec = lax.dot_general(xt, wg_ref[0], (((1,), (1,)), ((), ())),
                               preferred_element_type=jnp.float32)
        a = uvec * (gvec / (1.0 + jnp.exp(-gvec)))
        cvec = lax.dot_general(a, wd_ref[0], (((1,), (1,)), ((), ())),
                               preferred_element_type=jnp.float32)
        sl = pl.ds(rt * _TR, _TR)

        @pl.when(jj == 0)
        def _():
            out_ref[sl, :] = cvec

        @pl.when(jj > 0)
        def _():
            out_ref[sl, :] = out_ref[sl, :] + cvec


def _ffn(x_sorted, up, gate, down, rt_arr, jw_arr, vl_arr, ew_arr):
    grid_spec = pltpu.PrefetchScalarGridSpec(
        num_scalar_prefetch=4,
        grid=(_W,),
        in_specs=[
            pl.BlockSpec((_TR, _D), lambda w, rt, jw, vl, ew: (rt[w], 0)),
            pl.BlockSpec((1, _HBLK, _D),
                         lambda w, rt, jw, vl, ew: (ew[w], jw[w], 0)),
            pl.BlockSpec((1, _HBLK, _D),
                         lambda w, rt, jw, vl, ew: (ew[w], jw[w], 0)),
            pl.BlockSpec((1, _D, _HBLK),
                         lambda w, rt, jw, vl, ew: (ew[w], 0, jw[w])),
        ],
        out_specs=pl.BlockSpec((_ROWS, _D), lambda w, rt, jw, vl, ew: (0, 0)),
    )
    return pl.pallas_call(
        _ffn_body,
        grid_spec=grid_spec,
        out_shape=jax.ShapeDtypeStruct((_ROWS, _D), jnp.float32),
    )(rt_arr, jw_arr, vl_arr, ew_arr, x_sorted, up, gate, down)


def kernel(x, ffn_up_exps, ffn_gate_exps, ffn_down_exps, ffn_gate_inp):
    b, t, c = x.shape
    x_flat = x.reshape(b * t, c)

    topk_idx, topk_w = _router(x_flat, ffn_gate_inp)

    e_flat = topk_idx.reshape(-1)
    oh = (e_flat[:, None] == jnp.arange(_E, dtype=jnp.int32)[None, :])
    ranks_all = jnp.cumsum(oh.astype(jnp.int32), axis=0)
    rank = jnp.take_along_axis(ranks_all, e_flat[:, None], 1)[:, 0] - 1
    counts = ranks_all[-1]
    tiles_e = (counts + _TR - 1) // _TR
    cs_tiles = jnp.cumsum(tiles_e)
    nu = cs_tiles[-1]
    pos = (cs_tiles - tiles_e)[e_flat] * _TR + rank

    wv = jnp.arange(_W, dtype=jnp.int32)
    w_starts = _NH * (cs_tiles - tiles_e)
    ew_arr = jnp.sum(wv[:, None] >= _NH * cs_tiles[None, :], axis=1)
    ew_arr = jnp.clip(ew_arr, 0, _E - 1).astype(jnp.int32)
    te = jnp.maximum(tiles_e[ew_arr], 1)
    local = wv - w_starts[ew_arr]
    jw_arr = local // te
    rt_arr = (cs_tiles - tiles_e)[ew_arr] + local % te
    valid = wv < _NH * nu
    e_last = jnp.clip(jnp.sum(nu > cs_tiles), 0, _E - 1).astype(jnp.int32)
    ew_arr = jnp.where(valid, ew_arr, e_last).astype(jnp.int32)
    jw_arr = jnp.where(valid, jw_arr, _NH - 1).astype(jnp.int32)
    rt_arr = jnp.where(valid, rt_arr, jnp.maximum(nu - 1, 0)).astype(jnp.int32)
    vl_arr = valid.astype(jnp.int32)

    tok = (jnp.arange(_A, dtype=jnp.int32) // _K).reshape(_NW, _NCH, _RCH)
    pos3 = pos.astype(jnp.int32).reshape(_NW, _NCH, _RCH)
    x_sorted = _dispatch(x_flat, tok, pos3)

    y_rows = _ffn(x_sorted, ffn_up_exps, ffn_gate_exps, ffn_down_exps,
                  rt_arr, jw_arr, vl_arr, ew_arr)

    posc = pos.astype(jnp.int32).reshape(_NW, _NCH, 2 * _TCH)
    wc = jnp.broadcast_to(topk_w.reshape(_A)[:, None],
                          (_A, 16)).reshape(_NW, _NCH * 2 * _TCH, 16)
    y = _combine(y_rows, posc, wc)
    return y.reshape(b, t, c)


# speedup vs baseline: 1.4404x; 1.4404x over previous
import functools

import jax
import jax.numpy as jnp
from jax import lax
from jax.experimental import pallas as pl
from jax.experimental.pallas import tpu as pltpu
from jax.experimental.pallas import tpu_sc as plsc

_B = 1
_T = 2048
_BT = _B * _T
_D = 1024
_H = 4096
_E = 8
_K = 2

_TR = 256
_NTMAX = _BT * _K // _TR + _E
_ROWS = _NTMAX * _TR
_HBLK = 1024
_NH = _H // _HBLK
_W = _NH * _NTMAX

_NC = 2
_NS = 16
_NW = _NC * _NS
_A = _BT * _K
_APW = _A // _NW
_RCH = 32
_NCH = _APW // _RCH
_TPW = _BT // _NW
_TCH = 16
_LCH = _D // 16

_sc_mesh = plsc.VectorSubcoreMesh(core_axis_name="c", subcore_axis_name="s")


def _router_body(x_ref, g_ref, idx_ref, w_ref):
    logits = lax.dot_general(x_ref[...], g_ref[...],
                             (((1,), (1,)), ((), ())),
                             preferred_element_type=jnp.float32)
    col = lax.broadcasted_iota(jnp.int32, (_BT, _E), 1)
    m1 = jnp.max(logits, axis=1, keepdims=True)
    i1 = jnp.min(jnp.where(logits == m1, col, _E), axis=1, keepdims=True)
    l2 = jnp.where(col == i1, -jnp.inf, logits)
    m2 = jnp.max(l2, axis=1, keepdims=True)
    i2 = jnp.min(jnp.where(l2 == m2, col, _E), axis=1, keepdims=True)
    w1 = 1.0 / (1.0 + jnp.exp(m2 - m1))
    idx_ref[:, 0:1] = i1
    idx_ref[:, 1:2] = i2
    w_ref[:, 0:1] = w1
    w_ref[:, 1:2] = 1.0 - w1


def _router(x_flat, gate_inp):
    return pl.pallas_call(
        _router_body,
        out_shape=(jax.ShapeDtypeStruct((_BT, _K), jnp.int32),
                   jax.ShapeDtypeStruct((_BT, _K), jnp.float32)),
    )(x_flat, gate_inp)


@functools.partial(
    pl.kernel,
    out_type=jax.ShapeDtypeStruct((_ROWS, _D), jnp.float32),
    mesh=_sc_mesh,
    scratch_types=[
        pltpu.VMEM((_NCH, _RCH), jnp.int32),
        pltpu.VMEM((_NCH, _RCH), jnp.int32),
        pltpu.VMEM((2, _RCH, _D), jnp.float32),
        pltpu.SemaphoreType.DMA,
        pltpu.SemaphoreType.DMA,
    ],
)
def _dispatch(x_hbm, tok_hbm, pos_hbm, xs_hbm, tok_v, pos_v, rows_v, sem, sem2):
    wid = lax.axis_index("s") * _NC + lax.axis_index("c")
    pltpu.sync_copy(tok_hbm.at[wid], tok_v)
    pltpu.sync_copy(pos_hbm.at[wid], pos_v)
    gathers = [None] * _NCH
    scatters = [None] * _NCH
    gathers[0] = pltpu.async_copy(x_hbm.at[tok_v.at[0]], rows_v.at[0], sem)
    for c in range(_NCH):
        gathers[c].wait()
        scatters[c] = pltpu.async_copy(
            rows_v.at[c % 2], xs_hbm.at[pos_v.at[c]], sem2)
        if c + 1 < _NCH:
            if c >= 1:
                scatters[c - 1].wait()
            gathers[c + 1] = pltpu.async_copy(
                x_hbm.at[tok_v.at[c + 1]], rows_v.at[(c + 1) % 2], sem)
    if _NCH >= 2:
        scatters[_NCH - 2].wait()
    scatters[_NCH - 1].wait()


@functools.partial(
    pl.kernel,
    out_type=jax.ShapeDtypeStruct((_BT, _D), jnp.float32),
    mesh=_sc_mesh,
    scratch_types=[
        pltpu.VMEM((_NCH, 2 * _TCH), jnp.int32),
        pltpu.VMEM((_NCH * 2 * _TCH, 16), jnp.float32),
        pltpu.VMEM((2, 2 * _TCH, _D), jnp.float32),
        pltpu.VMEM((2, _TCH, _D), jnp.float32),
        pltpu.SemaphoreType.DMA,
        pltpu.SemaphoreType.DMA,
    ],
)
def _combine(y_hbm, pos_hbm, w_hbm, out_hbm, pos_v, w_v, rows_v, out_v, sem,
             sem2):
    wid = lax.axis_index("s") * _NC + lax.axis_index("c")
    pltpu.sync_copy(pos_hbm.at[wid], pos_v)
    pltpu.sync_copy(w_hbm.at[wid], w_v)
    gathers = [None] * _NCH
    stores = [None] * _NCH
    gathers[0] = pltpu.async_copy(y_hbm.at[pos_v.at[0]], rows_v.at[0], sem)
    for c in range(_NCH):
        gathers[c].wait()
        if c + 1 < _NCH:
            gathers[c + 1] = pltpu.async_copy(
                y_hbm.at[pos_v.at[c + 1]], rows_v.at[(c + 1) % 2], sem)
        if c >= 2:
            stores[c - 2].wait()

        def token_loop(j, carry, c=c):
            w0 = w_v[c * 2 * _TCH + 2 * j]
            w1 = w_v[c * 2 * _TCH + 2 * j + 1]
            for q in range(_LCH):
                sl = pl.ds(q * 16, 16)
                a0 = rows_v[c % 2, 2 * j, sl]
                a1 = rows_v[c % 2, 2 * j + 1, sl]
                out_v[c % 2, j, sl] = w0 * a0 + w1 * a1
            return carry

        lax.fori_loop(0, _TCH, token_loop, 0)
        stores[c] = pltpu.async_copy(
            out_v.at[c % 2], out_hbm.at[pl.ds(wid * _TPW + c * _TCH, _TCH)],
            sem2)
    if _NCH >= 2:
        stores[_NCH - 2].wait()
    stores[_NCH - 1].wait()


def _ffn_body(rt_ref, jw_ref, vl_ref, ew_ref, x_ref, wu_ref, wg_ref,
              wd_ref, out_ref):
    w = pl.program_id(0)
    rt = rt_ref[w]
    jj = jw_ref[w]

    @pl.when(vl_ref[w] == 1)
    def _():
        xt = x_ref[...]
        uvec = lax.dot_general(xt, wu_ref[0], (((1,), (1,)), ((), ())),
                               preferred_element_type=jnp.float32)
        gvec = lax.dot_general(xt, wg_ref[0], (((1,), (1,)), ((), ())),
                               preferred_element_type=jnp.float32)
        a = uvec * (gvec / (1.0 + jnp.exp(-gvec)))
        cvec = lax.dot_general(a, wd_ref[0], (((1,), (1,)), ((), ())),
                               preferred_element_type=jnp.float32)
        sl = pl.ds(rt * _TR, _TR)

        @pl.when(jj == 0)
        def _():
            out_ref[sl, :] = cvec

        @pl.when(jj > 0)
        def _():
            out_ref[sl, :] = out_ref[sl, :] + cvec


def _ffn(x_sorted, up, gate, down, rt_arr, jw_arr, vl_arr, ew_arr):
    grid_spec = pltpu.PrefetchScalarGridSpec(
        num_scalar_prefetch=4,
        grid=(_W,),
        in_specs=[
            pl.BlockSpec((_TR, _D), lambda w, rt, jw, vl, ew: (rt[w], 0)),
            pl.BlockSpec((1, _HBLK, _D),
                         lambda w, rt, jw, vl, ew: (ew[w], jw[w], 0)),
            pl.BlockSpec((1, _HBLK, _D),
                         lambda w, rt, jw, vl, ew: (ew[w], jw[w], 0)),
            pl.BlockSpec((1, _D, _HBLK),
                         lambda w, rt, jw, vl, ew: (ew[w], 0, jw[w])),
        ],
        out_specs=pl.BlockSpec((_ROWS, _D), lambda w, rt, jw, vl, ew: (0, 0)),
    )
    return pl.pallas_call(
        _ffn_body,
        grid_spec=grid_spec,
        out_shape=jax.ShapeDtypeStruct((_ROWS, _D), jnp.float32),
    )(rt_arr, jw_arr, vl_arr, ew_arr, x_sorted, up, gate, down)


def kernel(x, ffn_up_exps, ffn_gate_exps, ffn_down_exps, ffn_gate_inp):
    b, t, c = x.shape
    x_flat = x.reshape(b * t, c)

    topk_idx, topk_w = _router(x_flat, ffn_gate_inp)

    e_flat = topk_idx.reshape(-1)
    oh = (e_flat[:, None] == jnp.arange(_E, dtype=jnp.int32)[None, :])
    ranks_all = jnp.cumsum(oh.astype(jnp.int32), axis=0)
    rank = jnp.take_along_axis(ranks_all, e_flat[:, None], 1)[:, 0] - 1
    counts = ranks_all[-1]
    tiles_e = (counts + _TR - 1) // _TR
    cs_tiles = jnp.cumsum(tiles_e)
    nu = cs_tiles[-1]
    pos = (cs_tiles - tiles_e)[e_flat] * _TR + rank

    wv = jnp.arange(_W, dtype=jnp.int32)
    w_starts = _NH * (cs_tiles - tiles_e)
    ew_arr = jnp.sum(wv[:, None] >= _NH * cs_tiles[None, :], axis=1)
    ew_arr = jnp.clip(ew_arr, 0, _E - 1).astype(jnp.int32)
    te = jnp.maximum(tiles_e[ew_arr], 1)
    local = wv - w_starts[ew_arr]
    jw_arr = local // te
    rt_arr = (cs_tiles - tiles_e)[ew_arr] + local % te
    valid = wv < _NH * nu
    e_last = jnp.clip(jnp.sum(nu > cs_tiles), 0, _E - 1).astype(jnp.int32)
    ew_arr = jnp.where(valid, ew_arr, e_last).astype(jnp.int32)
    jw_arr = jnp.where(valid, jw_arr, _NH - 1).astype(jnp.int32)
    rt_arr = jnp.where(valid, rt_arr, jnp.maximum(nu - 1, 0)).astype(jnp.int32)
    vl_arr = valid.astype(jnp.int32)

    tok = (jnp.arange(_A, dtype=jnp.int32) // _K).reshape(_NW, _NCH, _RCH)
    pos3 = pos.astype(jnp.int32).reshape(_NW, _NCH, _RCH)
    x_sorted = _dispatch(x_flat, tok, pos3)

    y_rows = _ffn(x_sorted, ffn_up_exps, ffn_gate_exps, ffn_down_exps,
                  rt_arr, jw_arr, vl_arr, ew_arr)

    posc = pos.astype(jnp.int32).reshape(_NW, _NCH, 2 * _TCH)
    wc = jnp.broadcast_to(topk_w.reshape(_A)[:, None],
                          (_A, 16)).reshape(_NW, _NCH * 2 * _TCH, 16)
    y = _combine(y_rows, posc, wc)
    return y.reshape(b, t, c)


# speedup vs baseline: 1.5037x; 1.0440x over previous
import functools

import jax
import jax.numpy as jnp
from jax import lax
from jax.experimental import pallas as pl
from jax.experimental.pallas import tpu as pltpu
from jax.experimental.pallas import tpu_sc as plsc

_B = 1
_T = 2048
_BT = _B * _T
_D = 1024
_H = 4096
_E = 8
_K = 2

_TR = 256
_NTMAX = _BT * _K // _TR + _E
_ROWS = _NTMAX * _TR
_HBLK = 1024
_NH = _H // _HBLK
_W = _NH * _NTMAX

_NC = 2
_NS = 16
_NW = _NC * _NS
_A = _BT * _K
_APW = _A // _NW
_RCH = 32
_NCH = _APW // _RCH
_TPW = _BT // _NW
_TCH = 16
_LCH = _D // 16

_sc_mesh = plsc.VectorSubcoreMesh(core_axis_name="c", subcore_axis_name="s")


def _router_body(x_ref, g_ref, pos_ref, w_ref, rt_ref, jw_ref, ew_ref, vl_ref):
    logits = lax.dot_general(x_ref[...], g_ref[...],
                             (((1,), (1,)), ((), ())),
                             preferred_element_type=jnp.float32)
    col = lax.broadcasted_iota(jnp.int32, (_BT, _E), 1)
    m1 = jnp.max(logits, axis=1, keepdims=True)
    i1 = jnp.min(jnp.where(logits == m1, col, _E), axis=1, keepdims=True)
    l2 = jnp.where(col == i1, -jnp.inf, logits)
    m2 = jnp.max(l2, axis=1, keepdims=True)
    i2 = jnp.min(jnp.where(l2 == m2, col, _E), axis=1, keepdims=True)
    w1 = 1.0 / (1.0 + jnp.exp(m2 - m1))
    w_ref[:, 0:1] = w1
    w_ref[:, 1:2] = 1.0 - w1

    oh1 = jnp.where(col == i1, 1.0, 0.0)
    oh2 = jnp.where(col == i2, 1.0, 0.0)
    oht = oh1 + oh2
    tri = jnp.where(
        lax.broadcasted_iota(jnp.int32, (_BT, _BT), 0)
        > lax.broadcasted_iota(jnp.int32, (_BT, _BT), 1), 1.0, 0.0)
    exc = lax.dot_general(tri, oht, (((1,), (0,)), ((), ())),
                          preferred_element_type=jnp.float32)
    rank1 = jnp.sum(oh1 * exc, axis=1, keepdims=True)
    rank2 = jnp.sum(oh2 * exc, axis=1, keepdims=True)

    counts = jnp.sum(oht, axis=0, keepdims=True)
    tiles = jnp.floor((counts + (_TR - 1)) * (1.0 / _TR))
    tri_le = jnp.where(
        lax.broadcasted_iota(jnp.int32, (_E, _E), 0)
        <= lax.broadcasted_iota(jnp.int32, (_E, _E), 1), 1.0, 0.0)
    cs = lax.dot_general(tiles, tri_le, (((1,), (0,)), ((), ())),
                         preferred_element_type=jnp.float32)
    tstart = cs - tiles
    base1 = jnp.sum(oh1 * tstart, axis=1, keepdims=True) * _TR
    base2 = jnp.sum(oh2 * tstart, axis=1, keepdims=True) * _TR
    pos_ref[:, 0:1] = (base1 + rank1).astype(jnp.int32)
    pos_ref[:, 1:2] = (base2 + rank2).astype(jnp.int32)

    nu = jnp.max(cs)
    wv = lax.broadcasted_iota(jnp.int32, (_W, _E), 0)
    ei = lax.broadcasted_iota(jnp.int32, (_W, _E), 1)
    cs_i = cs.astype(jnp.int32)
    tiles_i = tiles.astype(jnp.int32)
    tstart_i = tstart.astype(jnp.int32)
    ew = jnp.sum(jnp.where(wv >= _NH * cs_i, 1, 0), axis=1, keepdims=True)
    ew = jnp.clip(ew, 0, _E - 1)
    ohe = jnp.where(ei == ew, 1, 0)
    te = jnp.maximum(jnp.sum(ohe * tiles_i, axis=1, keepdims=True), 1)
    ts_w = jnp.sum(ohe * tstart_i, axis=1, keepdims=True)
    local = wv[:, 0:1] - _NH * ts_w
    jw = local // te
    rt = ts_w + local % te
    nu_i = jnp.max(cs_i)
    valid = wv[:, 0:1] < _NH * nu_i
    e_last = jnp.sum(jnp.where(cs_i < nu_i, 1, 0), axis=1, keepdims=True)
    e_last = jnp.clip(e_last, 0, _E - 1)
    ew_ref[:, :] = jnp.where(valid, ew, e_last)
    jw_ref[:, :] = jnp.where(valid, jw, _NH - 1)
    rt_ref[:, :] = jnp.where(valid, rt, jnp.maximum(nu_i - 1, 0))
    vl_ref[:, :] = valid.astype(jnp.int32)


def _router(x_flat, gate_inp):
    return pl.pallas_call(
        _router_body,
        out_shape=(jax.ShapeDtypeStruct((_BT, _K), jnp.int32),
                   jax.ShapeDtypeStruct((_BT, _K), jnp.float32),
                   jax.ShapeDtypeStruct((_W, 1), jnp.int32),
                   jax.ShapeDtypeStruct((_W, 1), jnp.int32),
                   jax.ShapeDtypeStruct((_W, 1), jnp.int32),
                   jax.ShapeDtypeStruct((_W, 1), jnp.int32)),
    )(x_flat, gate_inp)


@functools.partial(
    pl.kernel,
    out_type=jax.ShapeDtypeStruct((_ROWS, _D), jnp.float32),
    mesh=_sc_mesh,
    scratch_types=[
        pltpu.VMEM((_NCH, _RCH), jnp.int32),
        pltpu.VMEM((_NCH, _RCH), jnp.int32),
        pltpu.VMEM((2, _RCH, _D), jnp.float32),
        pltpu.SemaphoreType.DMA,
        pltpu.SemaphoreType.DMA,
    ],
)
def _dispatch(x_hbm, tok_hbm, pos_hbm, xs_hbm, tok_v, pos_v, rows_v, sem, sem2):
    wid = lax.axis_index("s") * _NC + lax.axis_index("c")
    pltpu.sync_copy(tok_hbm.at[wid], tok_v)
    pltpu.sync_copy(pos_hbm.at[wid], pos_v)
    gathers = [None] * _NCH
    scatters = [None] * _NCH
    gathers[0] = pltpu.async_copy(x_hbm.at[tok_v.at[0]], rows_v.at[0], sem)
    for c in range(_NCH):
        gathers[c].wait()
        scatters[c] = pltpu.async_copy(
            rows_v.at[c % 2], xs_hbm.at[pos_v.at[c]], sem2)
        if c + 1 < _NCH:
            if c >= 1:
                scatters[c - 1].wait()
            gathers[c + 1] = pltpu.async_copy(
                x_hbm.at[tok_v.at[c + 1]], rows_v.at[(c + 1) % 2], sem)
    if _NCH >= 2:
        scatters[_NCH - 2].wait()
    scatters[_NCH - 1].wait()


@functools.partial(
    pl.kernel,
    out_type=jax.ShapeDtypeStruct((_BT, _D), jnp.float32),
    mesh=_sc_mesh,
    scratch_types=[
        pltpu.VMEM((_NCH, 2 * _TCH), jnp.int32),
        pltpu.VMEM((_NCH * 2 * _TCH, 16), jnp.float32),
        pltpu.VMEM((2, 2 * _TCH, _D), jnp.float32),
        pltpu.VMEM((2, _TCH, _D), jnp.float32),
        pltpu.SemaphoreType.DMA,
        pltpu.SemaphoreType.DMA,
    ],
)
def _combine(y_hbm, pos_hbm, w_hbm, out_hbm, pos_v, w_v, rows_v, out_v, sem,
             sem2):
    wid = lax.axis_index("s") * _NC + lax.axis_index("c")
    pltpu.sync_copy(pos_hbm.at[wid], pos_v)
    pltpu.sync_copy(w_hbm.at[wid], w_v)
    gathers = [None] * _NCH
    stores = [None] * _NCH
    gathers[0] = pltpu.async_copy(y_hbm.at[pos_v.at[0]], rows_v.at[0], sem)
    for c in range(_NCH):
        gathers[c].wait()
        if c + 1 < _NCH:
            gathers[c + 1] = pltpu.async_copy(
                y_hbm.at[pos_v.at[c + 1]], rows_v.at[(c + 1) % 2], sem)
        if c >= 2:
            stores[c - 2].wait()

        def token_loop(j, carry, c=c):
            w0 = w_v[c * 2 * _TCH + 2 * j]
            w1 = w_v[c * 2 * _TCH + 2 * j + 1]
            for q in range(_LCH):
                sl = pl.ds(q * 16, 16)
                a0 = rows_v[c % 2, 2 * j, sl]
                a1 = rows_v[c % 2, 2 * j + 1, sl]
                out_v[c % 2, j, sl] = w0 * a0 + w1 * a1
            return carry

        lax.fori_loop(0, _TCH, token_loop, 0)
        stores[c] = pltpu.async_copy(
            out_v.at[c % 2], out_hbm.at[pl.ds(wid * _TPW + c * _TCH, _TCH)],
            sem2)
    if _NCH >= 2:
        stores[_NCH - 2].wait()
    stores[_NCH - 1].wait()


def _ffn_body(rt_ref, jw_ref, vl_ref, ew_ref, x_ref, wu_ref, wg_ref,
              wd_ref, out_ref):
    w = pl.program_id(0)
    rt = rt_ref[w, 0]
    jj = jw_ref[w, 0]

    @pl.when(vl_ref[w, 0] == 1)
    def _():
        xt = x_ref[...]
        uvec = lax.dot_general(xt, wu_ref[0], (((1,), (1,)), ((), ())),
                               preferred_element_type=jnp.float32)
        gvec = lax.dot_general(xt, wg_ref[0], (((1,), (1,)), ((), ())),
                               preferred_element_type=jnp.float32)
        a = uvec * (gvec / (1.0 + jnp.exp(-gvec)))
        cvec = lax.dot_general(a, wd_ref[0], (((1,), (1,)), ((), ())),
                               preferred_element_type=jnp.float32)
        sl = pl.ds(rt * _TR, _TR)

        @pl.when(jj == 0)
        def _():
            out_ref[sl, :] = cvec

        @pl.when(jj > 0)
        def _():
            out_ref[sl, :] = out_ref[sl, :] + cvec


def _ffn(x_sorted, up, gate, down, rt_arr, jw_arr, vl_arr, ew_arr):
    grid_spec = pltpu.PrefetchScalarGridSpec(
        num_scalar_prefetch=4,
        grid=(_W,),
        in_specs=[
            pl.BlockSpec((_TR, _D), lambda w, rt, jw, vl, ew: (rt[w, 0], 0)),
            pl.BlockSpec((1, _HBLK, _D),
                         lambda w, rt, jw, vl, ew: (ew[w, 0], jw[w, 0], 0)),
            pl.BlockSpec((1, _HBLK, _D),
                         lambda w, rt, jw, vl, ew: (ew[w, 0], jw[w, 0], 0)),
            pl.BlockSpec((1, _D, _HBLK),
                         lambda w, rt, jw, vl, ew: (ew[w, 0], 0, jw[w, 0])),
        ],
        out_specs=pl.BlockSpec((_ROWS, _D), lambda w, rt, jw, vl, ew: (0, 0)),
    )
    return pl.pallas_call(
        _ffn_body,
        grid_spec=grid_spec,
        out_shape=jax.ShapeDtypeStruct((_ROWS, _D), jnp.float32),
    )(rt_arr, jw_arr, vl_arr, ew_arr, x_sorted, up, gate, down)


def kernel(x, ffn_up_exps, ffn_gate_exps, ffn_down_exps, ffn_gate_inp):
    b, t, c = x.shape
    x_flat = x.reshape(b * t, c)

    pos, topk_w, rt_arr, jw_arr, ew_arr, vl_arr = _router(x_flat, ffn_gate_inp)

    tok = (jnp.arange(_A, dtype=jnp.int32) // _K).reshape(_NW, _NCH, _RCH)
    pos3 = pos.reshape(_NW, _NCH, _RCH)
    x_sorted = _dispatch(x_flat, tok, pos3)

    y_rows = _ffn(x_sorted, ffn_up_exps, ffn_gate_exps, ffn_down_exps,
                  rt_arr, jw_arr, vl_arr, ew_arr)

    posc = pos.reshape(_NW, _NCH, 2 * _TCH)
    wc = jnp.broadcast_to(topk_w.reshape(_A)[:, None],
                          (_A, 16)).reshape(_NW, _NCH * 2 * _TCH, 16)
    y = _combine(y_rows, posc, wc)
    return y.reshape(b, t, c)
